# baseline (speedup 1.0000x reference)
import functools

import jax
import jax.numpy as jnp
from jax import lax
from jax.experimental import pallas as pl
from jax.experimental.pallas import tpu as pltpu
from jax.experimental.pallas import tpu_sc as plsc

BATCH = 4096
T = 200
D = 64
VOCAB = 1000000

NC = 2
NS = 16
NW = NC * NS

BATCH_PER_W = BATCH // NW
CHUNK_B = 2
CHUNK_ROWS = CHUNK_B * T
NCHUNK = BATCH_PER_W // CHUNK_B
SUB = 100
NSUB = CHUNK_ROWS // SUB
IDX_ROWS = BATCH_PER_W * T // SUB
OUT_W = 1024
OUT_ROWS_C = CHUNK_ROWS * D // OUT_W

_mesh = plsc.VectorSubcoreMesh(core_axis_name="c", subcore_axis_name="s")


@functools.partial(
    pl.kernel,
    mesh=_mesh,
    out_type=jax.ShapeDtypeStruct((BATCH * T * D // OUT_W, OUT_W), jnp.float32),
    compiler_params=pltpu.CompilerParams(use_tc_tiling_on_sc=False, skip_device_barrier=True),
    scratch_types=[
        pltpu.VMEM((NSUB, SUB), jnp.int32),
        pltpu.VMEM((NSUB, SUB), jnp.int32),
        pltpu.VMEM((NSUB, SUB), jnp.int32),
        pltpu.VMEM((NSUB, SUB), jnp.int32),
        pltpu.VMEM((CHUNK_ROWS, D), jnp.float32),
        pltpu.VMEM((CHUNK_ROWS, D), jnp.float32),
        pltpu.VMEM((OUT_ROWS_C, OUT_W), jnp.float32),
        pltpu.VMEM((OUT_ROWS_C, OUT_W), jnp.float32),
        pltpu.VMEM((T, D), jnp.float32),
        pltpu.SemaphoreType.DMA,
        pltpu.SemaphoreType.DMA,
        pltpu.SemaphoreType.DMA,
        pltpu.SemaphoreType.DMA,
        pltpu.SemaphoreType.DMA,
        pltpu.SemaphoreType.DMA,
    ],
)
def _emb_kernel(idx_hbm, tok_hbm, pos_hbm, out_hbm,
                ix0, ix1, ix2, ix3, rows0, rows1, st0, st1, pos_v,
                isem0, isem1, gsem0, gsem1, osem0, osem1):
    wid = lax.axis_index("s") * NC + lax.axis_index("c")
    ix = (ix0, ix1, ix2, ix3)
    rows = (rows0, rows1)
    stage = (st0, st1)
    isems = (isem0, isem1)
    gsems = (gsem0, gsem1)
    osems = (osem0, osem1)

    def start_idx(c, r):
        pltpu.async_copy(idx_hbm.at[wid, pl.ds(c * NSUB, NSUB)], ix[r],
                         isems[r % 2])

    def wait_idx(r):
        pltpu.make_async_copy(idx_hbm.at[wid, pl.ds(0, NSUB)], ix[r],
                              isems[r % 2]).wait()

    def start_gather(c, ring, par):
        r = ix[ring]
        for k in range(NSUB):
            pltpu.async_copy(
                tok_hbm.at[r.at[k]],
                rows[par].at[pl.ds(k * SUB, SUB)],
                gsems[par],
            )

    def wait_gather(par):
        pltpu.make_async_copy(
            tok_hbm.at[pl.ds(0, CHUNK_ROWS)], rows[par], gsems[par]
        ).wait()

    def add_pos(par):
        r = rows[par]
        s = stage[par]

        @plsc.parallel_loop(0, T, unroll=2)
        def _(t):
            for cc in range(D // 16):
                p = pos_v[t, pl.ds(cc * 16, 16)]
                for b in range(CHUNK_B):
                    flat = (b * T + t) * D + cc * 16
                    s[flat // OUT_W, pl.ds(flat % OUT_W, 16)] = (
                        r[b * T + t, pl.ds(cc * 16, 16)] + p
                    )

    def start_out(c, par):
        g = (wid * NCHUNK + c) * OUT_ROWS_C
        pltpu.async_copy(stage[par], out_hbm.at[pl.ds(g, OUT_ROWS_C)],
                         osems[par])

    def wait_out(par):
        pltpu.make_async_copy(
            stage[par], out_hbm.at[pl.ds(0, OUT_ROWS_C)], osems[par]
        ).wait()

    start_idx(0, 0)
    start_idx(1, 1)
    wait_idx(0)
    pltpu.sync_copy(pos_hbm, pos_v)
    start_gather(0, 0, 0)

    def loop_body(i, carry):
        for par4 in range(4):
            c = 4 * i + par4
            par = par4 % 2
            nxt = 1 - par

            @pl.when(c + 2 < NCHUNK)
            def _():
                start_idx(c + 2, (par4 + 2) % 4)

            @pl.when(c + 1 < NCHUNK)
            def _():
                wait_idx((par4 + 1) % 4)
                start_gather(c + 1, (par4 + 1) % 4, nxt)

            wait_gather(par)

            @pl.when(c >= 2)
            def _():
                wait_out(par)

            add_pos(par)
            start_out(c, par)
        return carry

    lax.fori_loop(0, NCHUNK // 4, loop_body, 0)
    wait_out(0)
    wait_out(1)


def kernel(idx, tok_table, pos_table):
    idx3 = idx.reshape(NW, IDX_ROWS, SUB).astype(jnp.int32)
    out = _emb_kernel(idx3, tok_table, pos_table)
    return out.reshape(BATCH, T, D)


# speedup vs baseline: 1.0024x; 1.0024x over previous
import functools

import jax
import jax.numpy as jnp
from jax import lax
from jax.experimental import pallas as pl
from jax.experimental.pallas import tpu as pltpu
from jax.experimental.pallas import tpu_sc as plsc

BATCH = 4096
T = 200
D = 64
VOCAB = 1000000

NC = 2
NS = 16
NW = NC * NS

BATCH_PER_W = BATCH // NW
CHUNK_B = 2
CHUNK_ROWS = CHUNK_B * T
NCHUNK = BATCH_PER_W // CHUNK_B
SUB = 100
NSUB = CHUNK_ROWS // SUB
IDX_ROWS = BATCH_PER_W * T // SUB
OUT_W = 1024
OUT_ROWS_C = CHUNK_ROWS * D // OUT_W

_mesh = plsc.VectorSubcoreMesh(core_axis_name="c", subcore_axis_name="s")


@functools.partial(
    pl.kernel,
    mesh=_mesh,
    out_type=jax.ShapeDtypeStruct((BATCH * T * D // OUT_W, OUT_W), jnp.float32),
    compiler_params=pltpu.CompilerParams(use_tc_tiling_on_sc=False),
    scratch_types=[
        pltpu.VMEM((NSUB, SUB), jnp.int32),
        pltpu.VMEM((NSUB, SUB), jnp.int32),
        pltpu.VMEM((NSUB, SUB), jnp.int32),
        pltpu.VMEM((NSUB, SUB), jnp.int32),
        pltpu.VMEM((CHUNK_ROWS, D), jnp.float32),
        pltpu.VMEM((CHUNK_ROWS, D), jnp.float32),
        pltpu.VMEM((OUT_ROWS_C, OUT_W), jnp.float32),
        pltpu.VMEM((OUT_ROWS_C, OUT_W), jnp.float32),
        pltpu.VMEM((T, D), jnp.float32),
        pltpu.SemaphoreType.DMA,
        pltpu.SemaphoreType.DMA,
        pltpu.SemaphoreType.DMA,
        pltpu.SemaphoreType.DMA,
        pltpu.SemaphoreType.DMA,
        pltpu.SemaphoreType.DMA,
    ],
)
def _emb_kernel(idx_hbm, tok_hbm, pos_hbm, out_hbm,
                ix0, ix1, ix2, ix3, rows0, rows1, st0, st1, pos_v,
                isem0, isem1, gsem0, gsem1, osem0, osem1):
    wid = lax.axis_index("s") * NC + lax.axis_index("c")
    ix = (ix0, ix1, ix2, ix3)
    rows = (rows0, rows1)
    stage = (st0, st1)
    isems = (isem0, isem1)
    gsems = (gsem0, gsem1)
    osems = (osem0, osem1)

    def start_idx(c, r):
        pltpu.async_copy(idx_hbm.at[wid, pl.ds(c * NSUB, NSUB)], ix[r],
                         isems[r % 2])

    def wait_idx(r):
        pltpu.make_async_copy(idx_hbm.at[wid, pl.ds(0, NSUB)], ix[r],
                              isems[r % 2]).wait()

    def start_gather(c, ring, par):
        r = ix[ring]
        for k in range(NSUB):
            pltpu.async_copy(
                tok_hbm.at[r.at[k]],
                rows[par].at[pl.ds(k * SUB, SUB)],
                gsems[par],
            )

    def wait_gather(par):
        pltpu.make_async_copy(
            tok_hbm.at[pl.ds(0, CHUNK_ROWS)], rows[par], gsems[par]
        ).wait()

    def add_pos(par):
        r = rows[par]
        s = stage[par]

        @plsc.parallel_loop(0, T, unroll=2)
        def _(t):
            for cc in range(D // 16):
                p = pos_v[t, pl.ds(cc * 16, 16)]
                for b in range(CHUNK_B):
                    flat = (b * T + t) * D + cc * 16
                    s[flat // OUT_W, pl.ds(flat % OUT_W, 16)] = (
                        r[b * T + t, pl.ds(cc * 16, 16)] + p
                    )

    def start_out(c, par):
        g = (wid * NCHUNK + c) * OUT_ROWS_C
        pltpu.async_copy(stage[par], out_hbm.at[pl.ds(g, OUT_ROWS_C)],
                         osems[par])

    def wait_out(par):
        pltpu.make_async_copy(
            stage[par], out_hbm.at[pl.ds(0, OUT_ROWS_C)], osems[par]
        ).wait()

    start_idx(0, 0)
    start_idx(1, 1)
    wait_idx(0)
    pltpu.sync_copy(pos_hbm, pos_v)
    start_gather(0, 0, 0)

    def loop_body(i, carry):
        for par4 in range(4):
            c = 4 * i + par4
            par = par4 % 2
            nxt = 1 - par

            @pl.when(c + 2 < NCHUNK)
            def _():
                start_idx(c + 2, (par4 + 2) % 4)

            @pl.when(c + 1 < NCHUNK)
            def _():
                wait_idx((par4 + 1) % 4)
                start_gather(c + 1, (par4 + 1) % 4, nxt)

            wait_gather(par)

            @pl.when(c >= 2)
            def _():
                wait_out(par)

            add_pos(par)
            start_out(c, par)
        return carry

    lax.fori_loop(0, NCHUNK // 4, loop_body, 0)
    wait_out(0)
    wait_out(1)


def kernel(idx, tok_table, pos_table):
    idx3 = idx.reshape(NW, IDX_ROWS, SUB).astype(jnp.int32)
    out = _emb_kernel(idx3, tok_table, pos_table)
    return out.reshape(BATCH, T, D)
